# baseline (speedup 1.0000x reference)
import functools

import jax
import jax.numpy as jnp
from jax import lax
from jax.experimental import pallas as pl
from jax.experimental.pallas import tpu as pltpu
from jax.experimental.pallas import tpu_sc as plsc

VOCAB = 1000000
DIM = 64
PDIM = 128
B = 4096
L = 200
N_ROWS = B * L

_info = plsc.get_sparse_core_info()
NC, NS = _info.num_cores, _info.num_subcores
NW = NC * NS
B_PER_W = B // NW

_TBLK = 2048
_TGRID = (VOCAB + _TBLK - 1) // _TBLK


def _transpose_table(table_t):
  def body(in_ref, out_ref):
    out_ref[:, :DIM] = in_ref[...].T

  return pl.pallas_call(
      body,
      grid=(_TGRID,),
      in_specs=[pl.BlockSpec((DIM, _TBLK), lambda i: (0, i))],
      out_specs=pl.BlockSpec((_TBLK, PDIM), lambda i: (i, 0)),
      out_shape=jax.ShapeDtypeStruct((VOCAB, PDIM), jnp.float32),
  )(table_t)


def _make_kernel():
  mesh = plsc.VectorSubcoreMesh(core_axis_name="c", subcore_axis_name="s")

  @functools.partial(
      pl.kernel,
      mesh=mesh,
      out_type=jax.ShapeDtypeStruct((L * DIM, B), jnp.float32),
      scratch_types=[
          pltpu.VMEM((B_PER_W, L), jnp.int32),
          pltpu.VMEM((B_PER_W,), jnp.int32),
          pltpu.VMEM((B_PER_W,), jnp.int32),
          pltpu.VMEM((B_PER_W, DIM), jnp.float32),
          pltpu.VMEM((B_PER_W, DIM), jnp.float32),
          pltpu.VMEM((DIM, B_PER_W), jnp.float32),
          pltpu.VMEM((DIM, B_PER_W), jnp.float32),
          pltpu.SemaphoreType.DMA,
          pltpu.SemaphoreType.DMA,
          pltpu.SemaphoreType.DMA,
          pltpu.SemaphoreType.DMA,
      ],
      compiler_params=pltpu.CompilerParams(use_tc_tiling_on_sc=False,
                                           needs_layout_passes=False),
  )
  def k(x_hbm, table_hbm, out_hbm, xb, ich0, ich1, rows0, rows1, t0, t1,
        g0, g1, s0, s1):
    wid = lax.axis_index("s") * NC + lax.axis_index("c")
    b0 = wid * B_PER_W
    pltpu.sync_copy(x_hbm.at[pl.ds(b0, B_PER_W)], xb)

    iota = lax.iota(jnp.int32, 16)

    def assemble(l, ich):
      ls = jnp.full((16,), l, jnp.int32)
      for m in range(8):
        v = plsc.load_gather(xb, [iota + 16 * m, ls])
        ich[pl.ds(16 * m, 16)] = v

    def start_gather(ich, rows, sem):
      pltpu.async_copy(table_hbm.at[ich], rows, sem)

    def wait_gather(ich, rows, sem):
      pltpu.make_async_copy(table_hbm.at[ich], rows, sem).wait()

    def transpose(rows, t):
      @pl.loop(0, DIM)
      def _(d):
        ds_ = jnp.full((16,), d, jnp.int32)
        for m in range(8):
          v = plsc.load_gather(rows, [iota + 16 * m, ds_])
          t[d, pl.ds(16 * m, 16)] = v

    def start_store(l, t, sem):
      pltpu.async_copy(t, out_hbm.at[pl.ds(l * DIM, DIM), pl.ds(b0, B_PER_W)],
                       sem)

    def wait_store(t, sem):
      pltpu.make_async_copy(t, out_hbm.at[pl.ds(0, DIM), pl.ds(b0, B_PER_W)],
                            sem).wait()

    assemble(0, ich0)
    start_gather(ich0, rows0, g0)
    assemble(1, ich1)
    start_gather(ich1, rows1, g1)

    @pl.loop(0, L // 2)
    def _(j):
      l0 = 2 * j
      wait_gather(ich0, rows0, g0)

      @pl.when(j > 0)
      def _():
        wait_store(t0, s0)

      transpose(rows0, t0)

      @pl.when(j < L // 2 - 1)
      def _():
        assemble(l0 + 2, ich0)
        start_gather(ich0, rows0, g0)

      start_store(l0, t0, s0)

      wait_gather(ich1, rows1, g1)

      @pl.when(j > 0)
      def _():
        wait_store(t1, s1)

      transpose(rows1, t1)

      @pl.when(j < L // 2 - 1)
      def _():
        assemble(l0 + 3, ich1)
        start_gather(ich1, rows1, g1)

      start_store(l0 + 1, t1, s1)

    wait_store(t0, s0)
    wait_store(t1, s1)

  return k


_gather = _make_kernel()


@jax.jit
def kernel(x, table):
  t128 = _transpose_table(table.T)
  t2 = t128.reshape(2 * VOCAB, DIM)
  p2 = _gather(x.astype(jnp.int32) * 2, t2)
  return jnp.transpose(p2).reshape(B, L, DIM)


# speedup vs baseline: 1.7215x; 1.7215x over previous
import functools

import jax
import jax.numpy as jnp
from jax import lax
from jax.experimental import pallas as pl
from jax.experimental.pallas import tpu as pltpu
from jax.experimental.pallas import tpu_sc as plsc
VOCAB = 1000000
DIM = 64
PDIM = 128
B = 4096
L = 200
N_ROWS = B * L

_info = plsc.get_sparse_core_info()
NC, NS = _info.num_cores, _info.num_subcores
NW = NC * NS
ROWS_PER_W = N_ROWS // NW
CHUNK = 512
N_CHUNKS = ROWS_PER_W // CHUNK
HALF = N_CHUNKS // 2

_TBLK = 2048
_TGRID = (VOCAB + _TBLK - 1) // _TBLK


def _transpose_table(table_t):
  def body(in_ref, out_ref):
    out_ref[:, :DIM] = in_ref[...].T

  return pl.pallas_call(
      body,
      grid=(_TGRID,),
      in_specs=[pl.BlockSpec((DIM, _TBLK), lambda i: (0, i))],
      out_specs=pl.BlockSpec((_TBLK, PDIM), lambda i: (i, 0)),
      out_shape=jax.ShapeDtypeStruct((VOCAB, PDIM), jnp.float32),
  )(table_t)


def _make_kernel():
  mesh = plsc.VectorSubcoreMesh(core_axis_name="c", subcore_axis_name="s")

  @functools.partial(
      pl.kernel,
      mesh=mesh,
      out_type=jax.ShapeDtypeStruct((N_ROWS, DIM), jnp.float32),
      scratch_types=[
          pltpu.VMEM((ROWS_PER_W,), jnp.int32),
          pltpu.VMEM((CHUNK, DIM), jnp.float32),
          pltpu.VMEM((CHUNK, DIM), jnp.float32),
          pltpu.SemaphoreType.DMA,
          pltpu.SemaphoreType.DMA,
          pltpu.SemaphoreType.DMA,
          pltpu.SemaphoreType.DMA,
      ],
      compiler_params=pltpu.CompilerParams(use_tc_tiling_on_sc=False),
  )
  def k(idx_hbm, table_hbm, out_hbm, idx_v, buf0, buf1, g0, g1, s0, s1):
    wid = lax.axis_index("s") * NC + lax.axis_index("c")
    base = wid * ROWS_PER_W
    pltpu.sync_copy(idx_hbm.at[pl.ds(base, ROWS_PER_W)], idx_v)

    def start_gather(chunk_i, buf, sem):
      pltpu.async_copy(
          table_hbm.at[idx_v.at[pl.ds(chunk_i * CHUNK, CHUNK)]], buf, sem)

    def wait_gather(buf, sem):
      pltpu.make_async_copy(table_hbm.at[idx_v.at[pl.ds(0, CHUNK)]], buf,
                            sem).wait()

    def start_store(chunk_i, buf, sem):
      pltpu.async_copy(buf, out_hbm.at[pl.ds(base + chunk_i * CHUNK, CHUNK)],
                       sem)

    def wait_store(buf, sem):
      pltpu.make_async_copy(buf, out_hbm.at[pl.ds(base, CHUNK)], sem).wait()

    start_gather(0, buf0, g0)

    @pl.loop(0, HALF)
    def _(j):
      i0 = 2 * j
      wait_gather(buf0, g0)

      @pl.when(j > 0)
      def _():
        wait_store(buf1, s1)

      start_gather(i0 + 1, buf1, g1)
      start_store(i0, buf0, s0)

      wait_gather(buf1, g1)

      @pl.when(j < HALF - 1)
      def _():
        wait_store(buf0, s0)
        start_gather(i0 + 2, buf0, g0)

      start_store(i0 + 1, buf1, s1)

    wait_store(buf0, s0)
    wait_store(buf1, s1)

  return k


_gather = _make_kernel()


@jax.jit
def kernel(x, table):
  idx = x.reshape(-1).astype(jnp.int32) * 2
  t128 = _transpose_table(table.T)
  t2 = t128.reshape(2 * VOCAB, DIM)
  flat = _gather(idx, t2)
  return flat.reshape(B, L, DIM)
